# baseline (device time: 17730 ns/iter reference)
import jax
import jax.numpy as jnp
from jax import lax
from jax.experimental import pallas as pl
from jax.experimental.pallas import tpu as pltpu

N_DEV = 8
SPLITS = 4


def kernel(A, B):
    m, k = A.shape
    k2, n = B.shape
    rows = m // N_DEV
    cols = n // SPLITS

    def body(a_ref, b_ref, out_ref, part_ref, staging, gat_ref,
             rs_send, rs_recv, ag_send, ag_recv):
        my = lax.axis_index("i")

        barrier_sem = pltpu.get_barrier_semaphore()
        for kk in range(1, N_DEV):
            peer = (my + kk) % N_DEV
            pl.semaphore_signal(
                barrier_sem,
                inc=1,
                device_id=(peer,),
                device_id_type=pl.DeviceIdType.MESH,
            )

        rs = [[] for _ in range(SPLITS)]

        def start_rs(q, kj):
            pj = (my + kj) % N_DEV
            rdma = pltpu.make_async_remote_copy(
                src_ref=part_ref.at[
                    pl.ds(pj * rows, rows), pl.ds(q * cols, cols)
                ],
                dst_ref=staging.at[kj, :, pl.ds(q * cols, cols)],
                send_sem=rs_send.at[q, kj],
                recv_sem=rs_recv.at[q, kj],
                device_id=(pj,),
                device_id_type=pl.DeviceIdType.MESH,
            )
            rdma.start()
            rs[q].append(rdma)

        for kk in range(1, N_DEV):
            peer = (my + kk) % N_DEV
            part_ref[pl.ds(peer * rows, rows)] = jnp.dot(
                a_ref[pl.ds(peer * rows, rows)],
                b_ref[...],
                preferred_element_type=jnp.float32,
            ).astype(jnp.bfloat16)
            if kk == 3:
                pl.semaphore_wait(barrier_sem, N_DEV - 1)
                for kj in (1, 2, 3):
                    start_rs(0, kj)
            elif kk > 3:
                start_rs(0, kk)
        for q in range(1, SPLITS):
            for kk in range(1, N_DEV):
                start_rs(q, kk)

        acc = jnp.dot(
            a_ref[pl.ds(my * rows, rows)],
            b_ref[...],
            preferred_element_type=jnp.float32,
        )

        ag = [[] for _ in range(SPLITS)]
        for q in range(SPLITS):
            acc_q = acc[:, q * cols:(q + 1) * cols]
            for kk in range(1, N_DEV):
                rs[q][kk - 1].wait_recv()
                acc_q = acc_q + staging[
                    kk, :, q * cols:(q + 1) * cols
                ].astype(jnp.float32)
            gat_ref[pl.ds(my * rows, rows), pl.ds(q * cols, cols)] = (
                acc_q.astype(jnp.bfloat16)
            )
            for kk in range(1, N_DEV):
                peer = (my + kk) % N_DEV
                rdma = pltpu.make_async_remote_copy(
                    src_ref=gat_ref.at[
                        pl.ds(my * rows, rows), pl.ds(q * cols, cols)
                    ],
                    dst_ref=gat_ref.at[
                        pl.ds(my * rows, rows), pl.ds(q * cols, cols)
                    ],
                    send_sem=ag_send.at[q, kk],
                    recv_sem=ag_recv.at[q, kk],
                    device_id=(peer,),
                    device_id_type=pl.DeviceIdType.MESH,
                )
                rdma.start()
                ag[q].append(rdma)
            out_ref[pl.ds(my * rows, rows), pl.ds(q * cols, cols)] = acc_q

        for q in range(SPLITS):
            for r in rs[q]:
                r.wait_send()
        for q in range(SPLITS):
            for kk in range(1, N_DEV):
                ag[q][kk - 1].wait_recv()
                src = (my + N_DEV - kk) % N_DEV
                out_ref[pl.ds(src * rows, rows), pl.ds(q * cols, cols)] = (
                    gat_ref[
                        pl.ds(src * rows, rows), pl.ds(q * cols, cols)
                    ].astype(jnp.float32)
                )
        for q in range(SPLITS):
            for r in ag[q]:
                r.wait_send()

    return pl.pallas_call(
        body,
        out_shape=jax.ShapeDtypeStruct((m, n), jnp.float32),
        in_specs=[
            pl.BlockSpec(memory_space=pltpu.VMEM),
            pl.BlockSpec(memory_space=pltpu.VMEM),
        ],
        out_specs=pl.BlockSpec(memory_space=pltpu.VMEM),
        scratch_shapes=[
            pltpu.VMEM((m, n), jnp.bfloat16),
            pltpu.VMEM((N_DEV, rows, n), jnp.bfloat16),
            pltpu.VMEM((m, n), jnp.bfloat16),
            pltpu.SemaphoreType.DMA((SPLITS, N_DEV)),
            pltpu.SemaphoreType.DMA((SPLITS, N_DEV)),
            pltpu.SemaphoreType.DMA((SPLITS, N_DEV)),
            pltpu.SemaphoreType.DMA((SPLITS, N_DEV)),
        ],
        compiler_params=pltpu.CompilerParams(collective_id=0),
    )(A, B)


# device time: 17615 ns/iter; 1.0065x vs baseline; 1.0065x over previous
import jax
import jax.numpy as jnp
from jax import lax
from jax.experimental import pallas as pl
from jax.experimental.pallas import tpu as pltpu

N_DEV = 8
SPLITS = 2


def kernel(A, B):
    m, k = A.shape
    k2, n = B.shape
    rows = m // N_DEV
    cols = n // SPLITS

    def body(a_ref, b_ref, out_ref, part_ref, staging, gat_ref,
             rs_send, rs_recv, ag_send, ag_recv):
        my = lax.axis_index("i")

        barrier_sem = pltpu.get_barrier_semaphore()
        for kk in range(1, N_DEV):
            peer = (my + kk) % N_DEV
            pl.semaphore_signal(
                barrier_sem,
                inc=1,
                device_id=(peer,),
                device_id_type=pl.DeviceIdType.MESH,
            )

        rs = [[] for _ in range(SPLITS)]

        def start_rs(q, kj):
            pj = (my + kj) % N_DEV
            rdma = pltpu.make_async_remote_copy(
                src_ref=part_ref.at[
                    pl.ds(pj * rows, rows), pl.ds(q * cols, cols)
                ],
                dst_ref=staging.at[kj, :, pl.ds(q * cols, cols)],
                send_sem=rs_send.at[q, kj],
                recv_sem=rs_recv.at[q, kj],
                device_id=(pj,),
                device_id_type=pl.DeviceIdType.MESH,
            )
            rdma.start()
            rs[q].append(rdma)

        for kk in range(1, N_DEV):
            peer = (my + kk) % N_DEV
            part_ref[pl.ds(peer * rows, rows)] = jnp.dot(
                a_ref[pl.ds(peer * rows, rows)],
                b_ref[...],
                preferred_element_type=jnp.float32,
            ).astype(jnp.bfloat16)
            if kk == 3:
                pl.semaphore_wait(barrier_sem, N_DEV - 1)
                for kj in (1, 2, 3):
                    start_rs(0, kj)
            elif kk > 3:
                start_rs(0, kk)
        for q in range(1, SPLITS):
            for kk in range(1, N_DEV):
                start_rs(q, kk)

        acc = jnp.dot(
            a_ref[pl.ds(my * rows, rows)],
            b_ref[...],
            preferred_element_type=jnp.float32,
        )

        ag = [[] for _ in range(SPLITS)]
        for q in range(SPLITS):
            acc_q = acc[:, q * cols:(q + 1) * cols]
            for kk in range(1, N_DEV):
                rs[q][kk - 1].wait_recv()
                acc_q = acc_q + staging[
                    kk, :, q * cols:(q + 1) * cols
                ].astype(jnp.float32)
            gat_ref[pl.ds(my * rows, rows), pl.ds(q * cols, cols)] = (
                acc_q.astype(jnp.bfloat16)
            )
            for kk in range(1, N_DEV):
                peer = (my + kk) % N_DEV
                rdma = pltpu.make_async_remote_copy(
                    src_ref=gat_ref.at[
                        pl.ds(my * rows, rows), pl.ds(q * cols, cols)
                    ],
                    dst_ref=gat_ref.at[
                        pl.ds(my * rows, rows), pl.ds(q * cols, cols)
                    ],
                    send_sem=ag_send.at[q, kk],
                    recv_sem=ag_recv.at[q, kk],
                    device_id=(peer,),
                    device_id_type=pl.DeviceIdType.MESH,
                )
                rdma.start()
                ag[q].append(rdma)
            out_ref[pl.ds(my * rows, rows), pl.ds(q * cols, cols)] = acc_q

        for q in range(SPLITS):
            for r in rs[q]:
                r.wait_send()
        for q in range(SPLITS):
            for kk in range(1, N_DEV):
                ag[q][kk - 1].wait_recv()
                src = (my + N_DEV - kk) % N_DEV
                out_ref[pl.ds(src * rows, rows), pl.ds(q * cols, cols)] = (
                    gat_ref[
                        pl.ds(src * rows, rows), pl.ds(q * cols, cols)
                    ].astype(jnp.float32)
                )
        for q in range(SPLITS):
            for r in ag[q]:
                r.wait_send()

    return pl.pallas_call(
        body,
        out_shape=jax.ShapeDtypeStruct((m, n), jnp.float32),
        in_specs=[
            pl.BlockSpec(memory_space=pltpu.VMEM),
            pl.BlockSpec(memory_space=pltpu.VMEM),
        ],
        out_specs=pl.BlockSpec(memory_space=pltpu.VMEM),
        scratch_shapes=[
            pltpu.VMEM((m, n), jnp.bfloat16),
            pltpu.VMEM((N_DEV, rows, n), jnp.bfloat16),
            pltpu.VMEM((m, n), jnp.bfloat16),
            pltpu.SemaphoreType.DMA((SPLITS, N_DEV)),
            pltpu.SemaphoreType.DMA((SPLITS, N_DEV)),
            pltpu.SemaphoreType.DMA((SPLITS, N_DEV)),
            pltpu.SemaphoreType.DMA((SPLITS, N_DEV)),
        ],
        compiler_params=pltpu.CompilerParams(collective_id=0),
    )(A, B)
